# baseline (speedup 1.0000x reference)
import functools

import jax
import jax.numpy as jnp
from jax import lax
from jax.experimental import pallas as pl
from jax.experimental.pallas import tpu as pltpu
from jax.experimental.pallas import tpu_sc as plsc

N_DICT = 1000000
EMB = 64
BATCH = 16384

NC = 2
NS = 16
NW = NC * NS

BLK = 32768
SCM = 10
SC_N = SCM * BLK
TC_N = N_DICT - SC_N
TC_GRID = -(-TC_N // BLK)

NPW = SC_N // NW
CBLK = 512
NBLK = NPW // CBLK
NSUB = CBLK // 128


def _matvec_body(t_ref, w_ref, z0_ref, z1_ref):
    p = jnp.dot(w_ref[...], t_ref[...], preferred_element_type=jnp.float32)
    z0_ref[...] = p[0]
    z1_ref[...] = p[1]


_matvec = pl.pallas_call(
    _matvec_body,
    grid=(TC_GRID,),
    in_specs=[
        pl.BlockSpec((EMB, BLK), lambda i: (0, SCM + i)),
        pl.BlockSpec((8, EMB), lambda i: (0, 0)),
    ],
    out_specs=[
        pl.BlockSpec((BLK,), lambda i: (i,)),
        pl.BlockSpec((BLK,), lambda i: (i,)),
    ],
    out_shape=[
        jax.ShapeDtypeStruct((TC_N,), jnp.float32),
        jax.ShapeDtypeStruct((TC_N,), jnp.float32),
    ],
    compiler_params=pltpu.CompilerParams(
        dimension_semantics=("arbitrary",),
    ),
)


def _scmv_body(t_hbm, w_hbm, z0_hbm, z1_hbm, buf, w_v, out0_v, out1_v,
               sem0, sem1):
    wid = lax.axis_index("s") * NC + lax.axis_index("c")
    vbase = wid * NPW

    pltpu.sync_copy(w_hbm, w_v)

    def fire(g, b):
        pltpu.async_copy(
            t_hbm.at[:, pl.ds(vbase + g * CBLK, CBLK)],
            buf.at[pl.ds(64 * b, 64)],
            (sem0, sem1)[b],
        )

    def wait(g, b):
        pltpu.make_async_copy(
            t_hbm.at[:, pl.ds(vbase + g * CBLK, CBLK)],
            buf.at[pl.ds(64 * b, 64)],
            (sem0, sem1)[b],
        ).wait()

    fire(0, 0)
    fire(1, 1)

    def step(gk, carry):
        g = gk // NSUB
        k = gk % NSUB
        odd = g % 2
        boff = odd * 64

        @pl.when((k == 0) & (odd == 0))
        def _():
            wait(g, 0)

        @pl.when((k == 0) & (odd == 1))
        def _():
            wait(g, 1)

        accs0 = [jnp.zeros((16,), jnp.float32) for _ in range(8)]
        accs1 = [jnp.zeros((16,), jnp.float32) for _ in range(8)]
        col = 128 * k
        for f in range(EMB):
            w0v = w_v[pl.ds(16 * f, 16)]
            w1v = w_v[pl.ds(1024 + 16 * f, 16)]
            for s in range(8):
                v = buf[boff + f, pl.ds(col + 16 * s, 16)]
                accs0[s] = accs0[s] + v * w0v
                accs1[s] = accs1[s] + v * w1v
        obase = g * CBLK + col
        for s in range(8):
            out0_v[pl.ds(obase + 16 * s, 16)] = accs0[s]
            out1_v[pl.ds(obase + 16 * s, 16)] = accs1[s]

        @pl.when((k == NSUB - 1) & (g + 2 < NBLK) & (odd == 0))
        def _():
            fire(g + 2, 0)

        @pl.when((k == NSUB - 1) & (g + 2 < NBLK) & (odd == 1))
        def _():
            fire(g + 2, 1)

        return carry

    lax.fori_loop(0, NBLK * NSUB, step, 0)

    pltpu.sync_copy(out0_v, z0_hbm.at[pl.ds(vbase, NPW)])
    pltpu.sync_copy(out1_v, z1_hbm.at[pl.ds(vbase, NPW)])


_scmv_call = functools.partial(
    pl.kernel,
    out_type=[
        jax.ShapeDtypeStruct((SC_N,), jnp.float32),
        jax.ShapeDtypeStruct((SC_N,), jnp.float32),
    ],
    scratch_types=[
        pltpu.VMEM((2 * EMB, CBLK), jnp.float32),
        pltpu.VMEM((16 * 2 * EMB,), jnp.float32),
        pltpu.VMEM((NPW,), jnp.float32),
        pltpu.VMEM((NPW,), jnp.float32),
        pltpu.SemaphoreType.DMA,
        pltpu.SemaphoreType.DMA,
    ],
    mesh=plsc.VectorSubcoreMesh(core_axis_name="c", subcore_axis_name="s"),
    compiler_params=pltpu.CompilerParams(
        needs_layout_passes=False, use_tc_tiling_on_sc=True
    ),
)(_scmv_body)


BPW = BATCH // NW
NCHUNK = 4
CHUNK = BPW // NCHUNK


def _gather_body(z0_hbm, z1_hbm, x0_hbm, x1_hbm, b_hbm, out_hbm,
                 idx0_v, idx1_v, g0_v, g1_v, b_v, out_v, sem):
    wid = lax.axis_index("s") * NC + lax.axis_index("c")
    base = wid * BPW

    pltpu.sync_copy(x0_hbm.at[pl.ds(base, BPW)], idx0_v)
    pltpu.sync_copy(x1_hbm.at[pl.ds(base, BPW)], idx1_v)
    pltpu.sync_copy(b_hbm, b_v)

    copies = []
    for j in range(NCHUNK):
        sl = pl.ds(j * CHUNK, CHUNK)
        copies.append(pltpu.async_copy(z0_hbm.at[idx0_v.at[sl]], g0_v.at[sl], sem))
        copies.append(pltpu.async_copy(z1_hbm.at[idx1_v.at[sl]], g1_v.at[sl], sem))
    for c in copies:
        c.wait()

    bias = b_v[...]
    for v in range(BPW // 16):
        sl = pl.ds(v * 16, 16)
        acc = g0_v[sl] + g1_v[sl] + bias
        out_v[sl] = 1.0 / (1.0 + jnp.exp(-acc))

    pltpu.sync_copy(out_v, out_hbm.at[pl.ds(base, BPW)])


_gather_call = functools.partial(
    pl.kernel,
    out_type=jax.ShapeDtypeStruct((BATCH,), jnp.float32),
    scratch_types=[
        pltpu.VMEM((BPW,), jnp.int32),
        pltpu.VMEM((BPW,), jnp.int32),
        pltpu.VMEM((BPW,), jnp.float32),
        pltpu.VMEM((BPW,), jnp.float32),
        pltpu.VMEM((16,), jnp.float32),
        pltpu.VMEM((BPW,), jnp.float32),
        pltpu.SemaphoreType.DMA,
    ],
    mesh=plsc.VectorSubcoreMesh(core_axis_name="c", subcore_axis_name="s"),
    compiler_params=pltpu.CompilerParams(
        needs_layout_passes=False, use_tc_tiling_on_sc=False
    ),
)(_gather_body)


def kernel(x, table, W, b):
    tt = table.T
    w8 = jnp.zeros((8, EMB), jnp.float32)
    w8 = w8.at[0].set(W[:EMB, 0]).at[1].set(W[EMB:, 0])
    z0t, z1t = _matvec(tt, w8)
    z0s, z1s = _scmv_call(tt, jnp.repeat(W[:, 0], 16))
    z0 = jnp.concatenate([z0s, z0t])
    z1 = jnp.concatenate([z1s, z1t])
    xi = x.astype(jnp.int32)
    bvec = jnp.full((16,), b[0], jnp.float32)
    out = _gather_call(z0, z1, xi[:, 0], xi[:, 1], bvec)
    return out.reshape(BATCH, 1)


# speedup vs baseline: 1.0968x; 1.0968x over previous
import functools

import jax
import jax.numpy as jnp
from jax import lax
from jax.experimental import pallas as pl
from jax.experimental.pallas import tpu as pltpu
from jax.experimental.pallas import tpu_sc as plsc

N_DICT = 1000000
EMB = 64
BATCH = 16384


BLK = 32768
GRID = -(-N_DICT // BLK)


def _matvec_body(t_ref, w_ref, z0_ref, z1_ref):
    p = jnp.dot(w_ref[...], t_ref[...], preferred_element_type=jnp.float32)
    z0_ref[...] = p[0]
    z1_ref[...] = p[1]


_matvec = pl.pallas_call(
    _matvec_body,
    grid=(GRID,),
    in_specs=[
        pl.BlockSpec((EMB, BLK), lambda i: (0, i)),
        pl.BlockSpec((2, EMB), lambda i: (0, 0)),
    ],
    out_specs=[
        pl.BlockSpec((BLK,), lambda i: (i,)),
        pl.BlockSpec((BLK,), lambda i: (i,)),
    ],
    out_shape=[
        jax.ShapeDtypeStruct((N_DICT,), jnp.float32),
        jax.ShapeDtypeStruct((N_DICT,), jnp.float32),
    ],
    compiler_params=pltpu.CompilerParams(
        dimension_semantics=("arbitrary",),
    ),
)


NC = 2
NS = 16
NW = NC * NS
BPW = BATCH // NW
NCHUNK = 4
CHUNK = BPW // NCHUNK


def _gather_body(z0_hbm, z1_hbm, x0_hbm, x1_hbm, b_hbm, out_hbm,
                 idx0_v, idx1_v, g0_v, g1_v, b_v, out_v, sem):
    wid = lax.axis_index("s") * NC + lax.axis_index("c")
    base = wid * BPW

    pltpu.sync_copy(x0_hbm.at[pl.ds(base, BPW)], idx0_v)
    pltpu.sync_copy(x1_hbm.at[pl.ds(base, BPW)], idx1_v)
    pltpu.sync_copy(b_hbm, b_v)

    copies = []
    for j in range(NCHUNK):
        sl = pl.ds(j * CHUNK, CHUNK)
        copies.append(pltpu.async_copy(z0_hbm.at[idx0_v.at[sl]], g0_v.at[sl], sem))
        copies.append(pltpu.async_copy(z1_hbm.at[idx1_v.at[sl]], g1_v.at[sl], sem))
    for c in copies:
        c.wait()

    bias = b_v[...]
    for v in range(BPW // 16):
        sl = pl.ds(v * 16, 16)
        acc = g0_v[sl] + g1_v[sl] + bias
        out_v[sl] = 1.0 / (1.0 + jnp.exp(-acc))

    pltpu.sync_copy(out_v, out_hbm.at[pl.ds(base, BPW)])


_gather_call = functools.partial(
    pl.kernel,
    out_type=jax.ShapeDtypeStruct((BATCH,), jnp.float32),
    scratch_types=[
        pltpu.VMEM((BPW,), jnp.int32),
        pltpu.VMEM((BPW,), jnp.int32),
        pltpu.VMEM((BPW,), jnp.float32),
        pltpu.VMEM((BPW,), jnp.float32),
        pltpu.VMEM((16,), jnp.float32),
        pltpu.VMEM((BPW,), jnp.float32),
        pltpu.SemaphoreType.DMA,
    ],
    mesh=plsc.VectorSubcoreMesh(core_axis_name="c", subcore_axis_name="s"),
    compiler_params=pltpu.CompilerParams(
        needs_layout_passes=False, use_tc_tiling_on_sc=False
    ),
)(_gather_body)


def kernel(x, table, W, b):
    z0, z1 = _matvec(table.T, W.reshape(2, EMB))
    xi = x.astype(jnp.int32)
    bvec = jnp.full((16,), b[0], jnp.float32)
    out = _gather_call(z0, z1, xi[:, 0], xi[:, 1], bvec)
    return out.reshape(BATCH, 1)


# speedup vs baseline: 1.1125x; 1.0143x over previous
import functools

import jax
import jax.numpy as jnp
from jax import lax
from jax.experimental import pallas as pl
from jax.experimental.pallas import tpu as pltpu
from jax.experimental.pallas import tpu_sc as plsc

N_DICT = 1000000
EMB = 64
BATCH = 16384


BLK = 32768
GRID = -(-N_DICT // BLK)


def _matvec_body(t_ref, w_ref, z0_ref, z1_ref):
    p = jnp.dot(w_ref[...], t_ref[...], preferred_element_type=jnp.float32)
    z0_ref[...] = p[0]
    z1_ref[...] = p[1]


_matvec = pl.pallas_call(
    _matvec_body,
    grid=(GRID,),
    in_specs=[
        pl.BlockSpec((EMB, BLK), lambda i: (0, i)),
        pl.BlockSpec((8, EMB), lambda i: (0, 0)),
    ],
    out_specs=[
        pl.BlockSpec((BLK,), lambda i: (i,)),
        pl.BlockSpec((BLK,), lambda i: (i,)),
    ],
    out_shape=[
        jax.ShapeDtypeStruct((N_DICT,), jnp.float32),
        jax.ShapeDtypeStruct((N_DICT,), jnp.float32),
    ],
    compiler_params=pltpu.CompilerParams(
        dimension_semantics=("arbitrary",),
    ),
)


NC = 2
NS = 16
NW = NC * NS
BPW = BATCH // NW
NCHUNK = 4
CHUNK = BPW // NCHUNK


def _gather_body(z0_hbm, z1_hbm, x0_hbm, x1_hbm, b_hbm, out_hbm,
                 idx0_v, idx1_v, g0_v, g1_v, b_v, out_v, sem):
    wid = lax.axis_index("s") * NC + lax.axis_index("c")
    base = wid * BPW

    pltpu.sync_copy(x0_hbm.at[pl.ds(base, BPW)], idx0_v)
    pltpu.sync_copy(x1_hbm.at[pl.ds(base, BPW)], idx1_v)
    pltpu.sync_copy(b_hbm, b_v)

    copies = []
    for j in range(NCHUNK):
        sl = pl.ds(j * CHUNK, CHUNK)
        copies.append(pltpu.async_copy(z0_hbm.at[idx0_v.at[sl]], g0_v.at[sl], sem))
        copies.append(pltpu.async_copy(z1_hbm.at[idx1_v.at[sl]], g1_v.at[sl], sem))
    for c in copies:
        c.wait()

    bias = b_v[...]
    for v in range(BPW // 16):
        sl = pl.ds(v * 16, 16)
        acc = g0_v[sl] + g1_v[sl] + bias
        out_v[sl] = 1.0 / (1.0 + jnp.exp(-acc))

    pltpu.sync_copy(out_v, out_hbm.at[pl.ds(base, BPW)])


_gather_call = functools.partial(
    pl.kernel,
    out_type=jax.ShapeDtypeStruct((BATCH,), jnp.float32),
    scratch_types=[
        pltpu.VMEM((BPW,), jnp.int32),
        pltpu.VMEM((BPW,), jnp.int32),
        pltpu.VMEM((BPW,), jnp.float32),
        pltpu.VMEM((BPW,), jnp.float32),
        pltpu.VMEM((16,), jnp.float32),
        pltpu.VMEM((BPW,), jnp.float32),
        pltpu.SemaphoreType.DMA,
    ],
    mesh=plsc.VectorSubcoreMesh(core_axis_name="c", subcore_axis_name="s"),
    compiler_params=pltpu.CompilerParams(
        needs_layout_passes=False, use_tc_tiling_on_sc=False
    ),
)(_gather_body)


def kernel(x, table, W, b):
    w8 = jnp.zeros((8, EMB), jnp.float32)
    w8 = w8.at[0].set(W[:EMB, 0]).at[1].set(W[EMB:, 0])
    z0, z1 = _matvec(table.T, w8)
    xi = x.astype(jnp.int32)
    bvec = jnp.full((16,), b[0], jnp.float32)
    out = _gather_call(z0, z1, xi[:, 0], xi[:, 1], bvec)
    return out.reshape(BATCH, 1)
